# baseline (speedup 1.0000x reference)
import functools

import jax
import jax.numpy as jnp
from jax import lax
from jax.experimental import pallas as pl
from jax.experimental.pallas import tpu as pltpu
from jax.experimental.pallas import tpu_sc as plsc

N = 10000
E = 320000
NFEAT = 128
NHID = 128
NCLASS = 64
HALF = 64

NC = 2
NS = 16
L = 16
NW = NC * NS

C = 128
NSLOT = 3
CHUNKS_PAD = 2592
E_PAD = CHUNKS_PAD * C
T1 = CHUNKS_PAD // NS
T2 = CHUNKS_PAD // NW
N_PAD = 10240
ROWS_PER_TILE = N_PAD // NS


def _make_sc_aggregate(col_split):
    T = T1 if col_split else T2
    NQ = T // NSLOT
    mesh = plsc.VectorSubcoreMesh(
        core_axis_name="c", subcore_axis_name="s",
        num_cores=NC, num_subcores=NS,
    )

    def body(*refs):
        if col_split:
            (sup, srca, srcb, tgt1, m1, out, acc, src_all, m_all,
             t0, t1, t2, r0, r1, r2,
             g0, g1, g2, s0, s1, s2, q0, q1, q2) = refs
        else:
            (sup, srca, tgt1, m1, init, out, acc, src_all, m_all,
             t0, t1, t2, r0, r1, r2,
             g0, g1, g2, s0, s1, s2, q0, q1, q2) = refs
        rows = (r0, r1, r2)
        tgtb = (t0, t1, t2)
        gsem = (g0, g1, g2)
        ssem = (s0, s1, s2)
        tsem = (q0, q1, q2)

        cid = lax.axis_index("c")
        sid = lax.axis_index("s")
        tile_base = sid * ROWS_PER_TILE
        if col_split:
            ebase = sid * T * C
        else:
            ebase = (sid * NC + cid) * T * C

        if col_split:
            @pl.when(cid == 0)
            def _():
                pltpu.sync_copy(srca.at[pl.ds(ebase, T * C)], src_all)

            @pl.when(cid != 0)
            def _():
                pltpu.sync_copy(srcb.at[pl.ds(ebase, T * C)], src_all)
        else:
            pltpu.sync_copy(srca.at[pl.ds(ebase, T * C)], src_all)
        pltpu.sync_copy(m1.at[pl.ds(ebase, T * C)], m_all)

        def zero_acc():
            zero = jnp.zeros((L,), jnp.float32)

            def zrow(r, carry):
                for j in range(HALF // L):
                    r0[r, pl.ds(j * L, L)] = zero
                return carry

            lax.fori_loop(0, C, zrow, 0)
            for i in range(ROWS_PER_TILE // C):
                pltpu.sync_copy(r0, acc.at[pl.ds(tile_base + i * C, C)])

        if col_split:
            zero_acc()
        else:
            @pl.when(cid == 0)
            def _():
                pltpu.sync_copy(init.at[pl.ds(tile_base, ROWS_PER_TILE)],
                                acc.at[pl.ds(tile_base, ROWS_PER_TILE)])

            @pl.when(cid != 0)
            def _():
                zero_acc()

        plsc.subcore_barrier()

        def gdesc(i, r):
            return pltpu.make_async_copy(
                sup.at[src_all.at[pl.ds(i * C, C)]], rows[r], gsem[r])

        def sdesc(i, r):
            return pltpu.make_async_copy(
                rows[r], acc.at[tgtb[r]], ssem[r])

        def tdesc(i, r):
            return pltpu.make_async_copy(
                tgt1.at[pl.ds(ebase + i * C, C)], tgtb[r], tsem[r])

        def fetch(i, r):
            gdesc(i, r).start()
            tdesc(i, r).start()

        def scale(i, r):
            rb = rows[r]

            @plsc.parallel_loop(0, C // L, unroll=4)
            def group(g):
                m16 = m_all[pl.ds(i * C + g * L, L)]
                for k in range(L):
                    mk = jnp.full((L,), m16[k])
                    e = g * L + k
                    for j in range(HALF // L):
                        sl = pl.ds(j * L, L)
                        rb[e, sl] = rb[e, sl] * mk

        fetch(0, 0)

        def qbody(q, carry):
            for r in range(NSLOT):
                i = q * NSLOT + r
                gdesc(i, r).wait()
                rp = (r + 1) % NSLOT
                if r == NSLOT - 1:
                    sdesc(i - 2, rp).wait()
                else:
                    @pl.when(q >= 1)
                    def _():
                        sdesc(i - 2, rp).wait()
                if r == NSLOT - 1:
                    @pl.when(q < NQ - 1)
                    def _():
                        fetch(i + 1, rp)
                else:
                    fetch(i + 1, rp)
                scale(i, r)
                tdesc(i, r).wait()
                pltpu.async_copy(rows[r], acc.at[tgtb[r]],
                                 ssem[r], add=True)
            return carry

        lax.fori_loop(0, NQ, qbody, 0)
        sdesc(T - 2, 1).wait()
        sdesc(T - 1, 2).wait()

        plsc.subcore_barrier()

        out_base = cid * N_PAD + tile_base
        pltpu.sync_copy(acc.at[pl.ds(tile_base, ROWS_PER_TILE)],
                        out.at[pl.ds(out_base, ROWS_PER_TILE)])

    return functools.partial(
        pl.kernel,
        out_type=jax.ShapeDtypeStruct((2 * N_PAD, HALF), jnp.float32),
        mesh=mesh,
        compiler_params=pltpu.CompilerParams(use_tc_tiling_on_sc=False),
        scratch_types=[
            pltpu.VMEM_SHARED((N_PAD, HALF), jnp.float32),
            pltpu.VMEM((T * C,), jnp.int32),
            pltpu.VMEM((T * C,), jnp.float32),
            pltpu.VMEM((C,), jnp.int32),
            pltpu.VMEM((C,), jnp.int32),
            pltpu.VMEM((C,), jnp.int32),
            pltpu.VMEM((C, HALF), jnp.float32),
            pltpu.VMEM((C, HALF), jnp.float32),
            pltpu.VMEM((C, HALF), jnp.float32),
            pltpu.SemaphoreType.DMA,
            pltpu.SemaphoreType.DMA,
            pltpu.SemaphoreType.DMA,
            pltpu.SemaphoreType.DMA,
            pltpu.SemaphoreType.DMA,
            pltpu.SemaphoreType.DMA,
            pltpu.SemaphoreType.DMA,
            pltpu.SemaphoreType.DMA,
            pltpu.SemaphoreType.DMA,
        ],
    )(body)


@functools.lru_cache(maxsize=None)
def _get_sc_l1():
    return _make_sc_aggregate(col_split=True)


@functools.lru_cache(maxsize=None)
def _get_sc_l2():
    return _make_sc_aggregate(col_split=False)


BM = 400
BM2 = 80
OFF2 = N_PAD // BM2


def _mm_split_body(x_ref, w_ref, b_ref, o_ref):
    s = (jnp.dot(x_ref[...], w_ref[...], preferred_element_type=jnp.float32)
         + b_ref[...])
    o_ref[...] = jnp.stack([s[:, :HALF], s[:, HALF:]], axis=0)


def _mm_split(x, W, b):
    return pl.pallas_call(
        _mm_split_body,
        grid=(N // BM,),
        in_specs=[
            pl.BlockSpec((BM, NFEAT), lambda i: (i, 0)),
            pl.BlockSpec((NFEAT, NHID), lambda i: (0, 0)),
            pl.BlockSpec((1, NHID), lambda i: (0, 0)),
        ],
        out_specs=pl.BlockSpec((2, BM, HALF), lambda i: (0, i, 0)),
        out_shape=jax.ShapeDtypeStruct((2, N, HALF), jnp.float32),
    )(x, W, b.reshape(1, NHID))


def _l2_body(pa_ref, pb_ref, w_ref, b_ref, ha_ref, s_ref):
    h = jnp.maximum(jnp.concatenate([pa_ref[...], pb_ref[...]], axis=1), 0.0)
    ha_ref[...] = h[:, :HALF]
    s_ref[...] = (
        jnp.dot(h, w_ref[...], preferred_element_type=jnp.float32)
        + b_ref[...]
    )


def _l2(parts, W2h, b2h):
    return pl.pallas_call(
        _l2_body,
        grid=(OFF2,),
        in_specs=[
            pl.BlockSpec((BM2, HALF), lambda i: (i, 0)),
            pl.BlockSpec((BM2, HALF), lambda i: (i + OFF2, 0)),
            pl.BlockSpec((NHID, HALF), lambda i: (0, 0)),
            pl.BlockSpec((1, HALF), lambda i: (0, 0)),
        ],
        out_specs=[
            pl.BlockSpec((BM2, HALF), lambda i: (i, 0)),
            pl.BlockSpec((BM2, HALF), lambda i: (i, 0)),
        ],
        out_shape=[
            jax.ShapeDtypeStruct((N_PAD, HALF), jnp.float32),
            jax.ShapeDtypeStruct((N_PAD, HALF), jnp.float32),
        ],
    )(parts, parts, W2h, b2h)


def _final_body(pa_ref, pb_ref, o_ref):
    v = pa_ref[...] + pb_ref[...]
    m = jnp.max(v, axis=1, keepdims=True)
    ex = jnp.exp(v - m)
    s = jnp.sum(ex, axis=1, keepdims=True)
    o_ref[...] = v - m - jnp.log(s)


def _final(parts):
    return pl.pallas_call(
        _final_body,
        grid=(N // BM2,),
        in_specs=[
            pl.BlockSpec((BM2, HALF), lambda i: (i, 0)),
            pl.BlockSpec((BM2, HALF), lambda i: (i + OFF2, 0)),
        ],
        out_specs=pl.BlockSpec((BM2, NCLASS), lambda i: (i, 0)),
        out_shape=jax.ShapeDtypeStruct((N, NCLASS), jnp.float32),
    )(parts, parts)


def kernel(x, src, tgt, Mtgt, W1, b1, W2, b2):
    npad = E_PAD - E
    pad_idx = jnp.arange(npad, dtype=jnp.int32) % N
    srca = jnp.concatenate([src.astype(jnp.int32), pad_idx])
    srcb = srca + N
    tgt1 = jnp.concatenate([tgt.astype(jnp.int32), pad_idx])
    m1 = jnp.concatenate([Mtgt, jnp.zeros((npad,), jnp.float32)])

    s1 = _mm_split(x, W1, b1).reshape(2 * N, HALF)
    p1 = _get_sc_l1()(s1, srca, srcb, tgt1, m1)
    ha, s2h = _l2(p1, W2[:, :HALF], b2[:HALF].reshape(1, HALF))
    p2 = _get_sc_l2()(s2h, srca, tgt1, m1, ha)
    return _final(p2)


# speedup vs baseline: 1.3674x; 1.3674x over previous
import functools

import jax
import jax.numpy as jnp
from jax import lax
from jax.experimental import pallas as pl
from jax.experimental.pallas import tpu as pltpu
from jax.experimental.pallas import tpu_sc as plsc

N = 10000
E = 320000
NFEAT = 128
NHID = 128
NCLASS = 64
HALF = 64

NC = 2
NS = 16
L = 16
NW = NC * NS

C = 128
NSLOT = 3
CHUNKS_PAD = 2592
E_PAD = CHUNKS_PAD * C
T1 = CHUNKS_PAD // NS
T2 = CHUNKS_PAD // NW
N_PAD = 10240
ROWS_PER_TILE = N_PAD // NS


def _make_sc_aggregate(col_split):
    T = T1 if col_split else T2
    NQ = T // NSLOT
    mesh = plsc.VectorSubcoreMesh(
        core_axis_name="c", subcore_axis_name="s",
        num_cores=NC, num_subcores=NS,
    )

    def body(*refs):
        if col_split:
            (sup, srca, srcb, tgt1, m1, out, acc, src_all, m_all,
             t0, t1, t2, r0, r1, r2,
             g0, g1, g2, s0, s1, s2, q0, q1, q2) = refs
        else:
            (sup, srca, tgt1, m1, init, out, acc, src_all, m_all,
             t0, t1, t2, r0, r1, r2,
             g0, g1, g2, s0, s1, s2, q0, q1, q2) = refs
        rows = (r0, r1, r2)
        tgtb = (t0, t1, t2)
        gsem = (g0, g1, g2)
        ssem = (s0, s1, s2)
        tsem = (q0, q1, q2)

        cid = lax.axis_index("c")
        sid = lax.axis_index("s")
        tile_base = sid * ROWS_PER_TILE
        if col_split:
            ebase = sid * T * C
        else:
            ebase = (sid * NC + cid) * T * C

        if col_split:
            @pl.when(cid == 0)
            def _():
                pltpu.sync_copy(srca.at[pl.ds(ebase, T * C)], src_all)

            @pl.when(cid != 0)
            def _():
                pltpu.sync_copy(srcb.at[pl.ds(ebase, T * C)], src_all)
        else:
            pltpu.sync_copy(srca.at[pl.ds(ebase, T * C)], src_all)
        pltpu.sync_copy(m1.at[pl.ds(ebase, T * C)], m_all)

        def zero_acc():
            zero = jnp.zeros((L,), jnp.float32)

            def zrow(r, carry):
                for j in range(HALF // L):
                    r0[r, pl.ds(j * L, L)] = zero
                return carry

            lax.fori_loop(0, C, zrow, 0)
            for i in range(ROWS_PER_TILE // C):
                pltpu.sync_copy(r0, acc.at[pl.ds(tile_base + i * C, C)])

        if col_split:
            zero_acc()
        else:
            @pl.when(cid == 0)
            def _():
                pltpu.sync_copy(init.at[pl.ds(tile_base, ROWS_PER_TILE)],
                                acc.at[pl.ds(tile_base, ROWS_PER_TILE)])

            @pl.when(cid != 0)
            def _():
                zero_acc()

        plsc.subcore_barrier()

        def gdesc(i, r):
            return pltpu.make_async_copy(
                sup.at[src_all.at[pl.ds(i * C, C)]], rows[r], gsem[r])

        def sdesc(i, r):
            return pltpu.make_async_copy(
                rows[r], acc.at[tgtb[r]], ssem[r])

        def tdesc(i, r):
            return pltpu.make_async_copy(
                tgt1.at[pl.ds(ebase + i * C, C)], tgtb[r], tsem[r])

        def fetch(i, r):
            gdesc(i, r).start()
            tdesc(i, r).start()

        def scale(i, r):
            rb = rows[r]

            @plsc.parallel_loop(0, C // L, unroll=4)
            def group(g):
                m16 = m_all[pl.ds(i * C + g * L, L)]
                for k in range(L):
                    mk = jnp.full((L,), m16[k])
                    e = g * L + k
                    for j in range(HALF // L):
                        sl = pl.ds(j * L, L)
                        rb[e, sl] = rb[e, sl] * mk

        fetch(0, 0)

        def qbody(q, carry):
            for r in range(NSLOT):
                i = q * NSLOT + r
                gdesc(i, r).wait()
                rp = (r + 1) % NSLOT
                if r == NSLOT - 1:
                    sdesc(i - 2, rp).wait()
                else:
                    @pl.when(q >= 1)
                    def _():
                        sdesc(i - 2, rp).wait()
                if r == NSLOT - 1:
                    @pl.when(q < NQ - 1)
                    def _():
                        fetch(i + 1, rp)
                else:
                    fetch(i + 1, rp)
                scale(i, r)
                tdesc(i, r).wait()
                pltpu.async_copy(rows[r], acc.at[tgtb[r]],
                                 ssem[r], add=True)
            return carry

        lax.fori_loop(0, NQ, qbody, 0)
        sdesc(T - 2, 1).wait()
        sdesc(T - 1, 2).wait()

        plsc.subcore_barrier()

        pltpu.sync_copy(acc.at[pl.ds(tile_base, ROWS_PER_TILE)],
                        out.at[cid, pl.ds(tile_base, ROWS_PER_TILE)])

    return functools.partial(
        pl.kernel,
        out_type=jax.ShapeDtypeStruct((2, N_PAD, HALF), jnp.float32),
        mesh=mesh,
        compiler_params=pltpu.CompilerParams(use_tc_tiling_on_sc=False),
        scratch_types=[
            pltpu.VMEM_SHARED((N_PAD, HALF), jnp.float32),
            pltpu.VMEM((T * C,), jnp.int32),
            pltpu.VMEM((T * C,), jnp.float32),
            pltpu.VMEM((C,), jnp.int32),
            pltpu.VMEM((C,), jnp.int32),
            pltpu.VMEM((C,), jnp.int32),
            pltpu.VMEM((C, HALF), jnp.float32),
            pltpu.VMEM((C, HALF), jnp.float32),
            pltpu.VMEM((C, HALF), jnp.float32),
            pltpu.SemaphoreType.DMA,
            pltpu.SemaphoreType.DMA,
            pltpu.SemaphoreType.DMA,
            pltpu.SemaphoreType.DMA,
            pltpu.SemaphoreType.DMA,
            pltpu.SemaphoreType.DMA,
            pltpu.SemaphoreType.DMA,
            pltpu.SemaphoreType.DMA,
            pltpu.SemaphoreType.DMA,
        ],
    )(body)


@functools.lru_cache(maxsize=None)
def _get_sc_l1():
    return _make_sc_aggregate(col_split=True)


@functools.lru_cache(maxsize=None)
def _get_sc_l2():
    return _make_sc_aggregate(col_split=False)


BMM = 2000
BML = 2048
BMF = 2000


def _mm_split_body(x_ref, w_ref, b_ref, o_ref):
    o_ref[...] = (
        jnp.dot(x_ref[...], w_ref[...], preferred_element_type=jnp.float32)
        + b_ref[0]
    )


def _mm_split(x, Wst, bst):
    nb = N // BMM
    return pl.pallas_call(
        _mm_split_body,
        grid=(2, nb),
        in_specs=[
            pl.BlockSpec((BMM, NFEAT), lambda c, i: (i, 0)),
            pl.BlockSpec((NFEAT, HALF), lambda c, i: (c, 0)),
            pl.BlockSpec((1, 1, HALF), lambda c, i: (c, 0, 0)),
        ],
        out_specs=pl.BlockSpec((BMM, HALF), lambda c, i: (c * nb + i, 0)),
        out_shape=jax.ShapeDtypeStruct((2 * N, HALF), jnp.float32),
    )(x, Wst, bst)


def _l2_body(pa_ref, pb_ref, w_ref, b_ref, ha_ref, s_ref):
    ra = jnp.maximum(pa_ref[0], 0.0)
    rb = jnp.maximum(pb_ref[0], 0.0)
    ha_ref[...] = ra
    s_ref[...] = (
        jnp.dot(ra, w_ref[:HALF, :], preferred_element_type=jnp.float32)
        + jnp.dot(rb, w_ref[HALF:, :], preferred_element_type=jnp.float32)
        + b_ref[...]
    )


def _l2(parts, W2h, b2h):
    return pl.pallas_call(
        _l2_body,
        grid=(N_PAD // BML,),
        in_specs=[
            pl.BlockSpec((1, BML, HALF), lambda i: (0, i, 0)),
            pl.BlockSpec((1, BML, HALF), lambda i: (1, i, 0)),
            pl.BlockSpec((NHID, HALF), lambda i: (0, 0)),
            pl.BlockSpec((1, HALF), lambda i: (0, 0)),
        ],
        out_specs=[
            pl.BlockSpec((BML, HALF), lambda i: (i, 0)),
            pl.BlockSpec((BML, HALF), lambda i: (i, 0)),
        ],
        out_shape=[
            jax.ShapeDtypeStruct((N_PAD, HALF), jnp.float32),
            jax.ShapeDtypeStruct((N_PAD, HALF), jnp.float32),
        ],
    )(parts, parts, W2h, b2h)


def _final_body(pa_ref, pb_ref, o_ref):
    v = pa_ref[0] + pb_ref[0]
    m = jnp.max(v, axis=1, keepdims=True)
    ex = jnp.exp(v - m)
    s = jnp.sum(ex, axis=1, keepdims=True)
    o_ref[...] = v - m - jnp.log(s)


def _final(parts):
    return pl.pallas_call(
        _final_body,
        grid=(N // BMF,),
        in_specs=[
            pl.BlockSpec((1, BMF, HALF), lambda i: (0, i, 0)),
            pl.BlockSpec((1, BMF, HALF), lambda i: (1, i, 0)),
        ],
        out_specs=pl.BlockSpec((BMF, NCLASS), lambda i: (i, 0)),
        out_shape=jax.ShapeDtypeStruct((N, NCLASS), jnp.float32),
    )(parts, parts)


def kernel(x, src, tgt, Mtgt, W1, b1, W2, b2):
    npad = E_PAD - E
    pad_idx = jnp.arange(npad, dtype=jnp.int32) % N
    srca = jnp.concatenate([src.astype(jnp.int32), pad_idx])
    srcb = srca + N
    tgt1 = jnp.concatenate([tgt.astype(jnp.int32), pad_idx])
    m1 = jnp.concatenate([Mtgt, jnp.zeros((npad,), jnp.float32)])

    w1st = jnp.concatenate([W1[:, :HALF], W1[:, HALF:]], axis=0)
    b1st = jnp.stack([b1[:HALF], b1[HALF:]]).reshape(2, 1, HALF)
    s1 = _mm_split(x, w1st, b1st)
    p1 = _get_sc_l1()(s1, srca, srcb, tgt1, m1)
    ha, s2h = _l2(p1, W2[:, :HALF], b2[:HALF].reshape(1, HALF))
    p2 = _get_sc_l2()(s2h, srca, tgt1, m1, ha)
    return _final(p2)


# speedup vs baseline: 1.3810x; 1.0099x over previous
import functools

import jax
import jax.numpy as jnp
from jax import lax
from jax.experimental import pallas as pl
from jax.experimental.pallas import tpu as pltpu
from jax.experimental.pallas import tpu_sc as plsc

N = 10000
E = 320000
NFEAT = 128
NHID = 128
NCLASS = 64
HALF = 64

NC = 2
NS = 16
L = 16
NW = NC * NS

C = 128
NSLOT = 4
CHUNKS_PAD = 2560
E_PAD = CHUNKS_PAD * C
T1 = CHUNKS_PAD // NS
T2 = CHUNKS_PAD // NW
N_PAD = 10240
ROWS_PER_TILE = N_PAD // NS


def _make_sc_aggregate(col_split):
    T = T1 if col_split else T2
    NQ = T // NSLOT
    mesh = plsc.VectorSubcoreMesh(
        core_axis_name="c", subcore_axis_name="s",
        num_cores=NC, num_subcores=NS,
    )

    def body(*refs):
        if col_split:
            (sup, srca, srcb, tgt1, m1, out, acc, src_all, m_all,
             *rest) = refs
        else:
            (sup, srca, tgt1, m1, init, out, acc, src_all, m_all,
             *rest) = refs
        tgtb = tuple(rest[0:NSLOT])
        rows = tuple(rest[NSLOT:2 * NSLOT])
        gsem = tuple(rest[2 * NSLOT:3 * NSLOT])
        ssem = tuple(rest[3 * NSLOT:4 * NSLOT])
        tsem = tuple(rest[4 * NSLOT:5 * NSLOT])
        r0 = rows[0]

        cid = lax.axis_index("c")
        sid = lax.axis_index("s")
        tile_base = sid * ROWS_PER_TILE
        if col_split:
            ebase = sid * T * C
        else:
            ebase = (sid * NC + cid) * T * C

        if col_split:
            @pl.when(cid == 0)
            def _():
                pltpu.sync_copy(srca.at[pl.ds(ebase, T * C)], src_all)

            @pl.when(cid != 0)
            def _():
                pltpu.sync_copy(srcb.at[pl.ds(ebase, T * C)], src_all)
        else:
            pltpu.sync_copy(srca.at[pl.ds(ebase, T * C)], src_all)
        pltpu.sync_copy(m1.at[pl.ds(ebase, T * C)], m_all)

        def zero_acc():
            zero = jnp.zeros((L,), jnp.float32)

            def zrow(r, carry):
                for j in range(HALF // L):
                    r0[r, pl.ds(j * L, L)] = zero
                return carry

            lax.fori_loop(0, C, zrow, 0)
            for i in range(ROWS_PER_TILE // C):
                pltpu.sync_copy(r0, acc.at[pl.ds(tile_base + i * C, C)])

        if col_split:
            zero_acc()
        else:
            @pl.when(cid == 0)
            def _():
                pltpu.sync_copy(init.at[pl.ds(tile_base, ROWS_PER_TILE)],
                                acc.at[pl.ds(tile_base, ROWS_PER_TILE)])

            @pl.when(cid != 0)
            def _():
                zero_acc()

        plsc.subcore_barrier()

        def gdesc(i, r):
            return pltpu.make_async_copy(
                sup.at[src_all.at[pl.ds(i * C, C)]], rows[r], gsem[r])

        def sdesc(i, r):
            return pltpu.make_async_copy(
                rows[r], acc.at[tgtb[r]], ssem[r])

        def tdesc(i, r):
            return pltpu.make_async_copy(
                tgt1.at[pl.ds(ebase + i * C, C)], tgtb[r], tsem[r])

        def fetch(i, r):
            gdesc(i, r).start()
            tdesc(i, r).start()

        def scale(i, r):
            rb = rows[r]

            @plsc.parallel_loop(0, C // L, unroll=4)
            def group(g):
                m16 = m_all[pl.ds(i * C + g * L, L)]
                for k in range(L):
                    mk = jnp.full((L,), m16[k])
                    e = g * L + k
                    for j in range(HALF // L):
                        sl = pl.ds(j * L, L)
                        rb[e, sl] = rb[e, sl] * mk

        fetch(0, 0)

        def qbody(q, carry):
            for r in range(NSLOT):
                i = q * NSLOT + r
                gdesc(i, r).wait()
                rp = (r + 1) % NSLOT
                if r == NSLOT - 1:
                    sdesc(i + 1 - NSLOT, rp).wait()
                else:
                    @pl.when(q >= 1)
                    def _():
                        sdesc(i + 1 - NSLOT, rp).wait()
                if r == NSLOT - 1:
                    @pl.when(q < NQ - 1)
                    def _():
                        fetch(i + 1, rp)
                else:
                    fetch(i + 1, rp)
                scale(i, r)
                tdesc(i, r).wait()
                pltpu.async_copy(rows[r], acc.at[tgtb[r]],
                                 ssem[r], add=True)
            return carry

        lax.fori_loop(0, NQ, qbody, 0)
        for k in range(NSLOT - 1, 0, -1):
            c = T - k
            sdesc(c, c % NSLOT).wait()

        plsc.subcore_barrier()

        pltpu.sync_copy(acc.at[pl.ds(tile_base, ROWS_PER_TILE)],
                        out.at[cid, pl.ds(tile_base, ROWS_PER_TILE)])

    return functools.partial(
        pl.kernel,
        out_type=jax.ShapeDtypeStruct((2, N_PAD, HALF), jnp.float32),
        mesh=mesh,
        compiler_params=pltpu.CompilerParams(use_tc_tiling_on_sc=False),
        scratch_types=(
            [
                pltpu.VMEM_SHARED((N_PAD, HALF), jnp.float32),
                pltpu.VMEM((T * C,), jnp.int32),
                pltpu.VMEM((T * C,), jnp.float32),
            ]
            + [pltpu.VMEM((C,), jnp.int32) for _ in range(NSLOT)]
            + [pltpu.VMEM((C, HALF), jnp.float32) for _ in range(NSLOT)]
            + [pltpu.SemaphoreType.DMA for _ in range(3 * NSLOT)]
        ),
    )(body)


@functools.lru_cache(maxsize=None)
def _get_sc_l1():
    return _make_sc_aggregate(col_split=True)


@functools.lru_cache(maxsize=None)
def _get_sc_l2():
    return _make_sc_aggregate(col_split=False)


BMM = 2000
BML = 2048
BMF = 2000


def _mm_split_body(x_ref, w_ref, b_ref, o_ref):
    o_ref[...] = (
        jnp.dot(x_ref[...], w_ref[...], preferred_element_type=jnp.float32)
        + b_ref[0]
    )


def _mm_split(x, Wst, bst):
    nb = N // BMM
    return pl.pallas_call(
        _mm_split_body,
        grid=(2, nb),
        in_specs=[
            pl.BlockSpec((BMM, NFEAT), lambda c, i: (i, 0)),
            pl.BlockSpec((NFEAT, HALF), lambda c, i: (c, 0)),
            pl.BlockSpec((1, 1, HALF), lambda c, i: (c, 0, 0)),
        ],
        out_specs=pl.BlockSpec((BMM, HALF), lambda c, i: (c * nb + i, 0)),
        out_shape=jax.ShapeDtypeStruct((2 * N, HALF), jnp.float32),
    )(x, Wst, bst)


def _l2_body(pa_ref, pb_ref, w_ref, b_ref, ha_ref, s_ref):
    ra = jnp.maximum(pa_ref[0], 0.0)
    rb = jnp.maximum(pb_ref[0], 0.0)
    ha_ref[...] = ra
    s_ref[...] = (
        jnp.dot(ra, w_ref[:HALF, :], preferred_element_type=jnp.float32)
        + jnp.dot(rb, w_ref[HALF:, :], preferred_element_type=jnp.float32)
        + b_ref[...]
    )


def _l2(parts, W2h, b2h):
    return pl.pallas_call(
        _l2_body,
        grid=(N_PAD // BML,),
        in_specs=[
            pl.BlockSpec((1, BML, HALF), lambda i: (0, i, 0)),
            pl.BlockSpec((1, BML, HALF), lambda i: (1, i, 0)),
            pl.BlockSpec((NHID, HALF), lambda i: (0, 0)),
            pl.BlockSpec((1, HALF), lambda i: (0, 0)),
        ],
        out_specs=[
            pl.BlockSpec((BML, HALF), lambda i: (i, 0)),
            pl.BlockSpec((BML, HALF), lambda i: (i, 0)),
        ],
        out_shape=[
            jax.ShapeDtypeStruct((N_PAD, HALF), jnp.float32),
            jax.ShapeDtypeStruct((N_PAD, HALF), jnp.float32),
        ],
    )(parts, parts, W2h, b2h)


def _final_body(pa_ref, pb_ref, o_ref):
    v = pa_ref[0] + pb_ref[0]
    m = jnp.max(v, axis=1, keepdims=True)
    ex = jnp.exp(v - m)
    s = jnp.sum(ex, axis=1, keepdims=True)
    o_ref[...] = v - m - jnp.log(s)


def _final(parts):
    return pl.pallas_call(
        _final_body,
        grid=(N // BMF,),
        in_specs=[
            pl.BlockSpec((1, BMF, HALF), lambda i: (0, i, 0)),
            pl.BlockSpec((1, BMF, HALF), lambda i: (1, i, 0)),
        ],
        out_specs=pl.BlockSpec((BMF, NCLASS), lambda i: (i, 0)),
        out_shape=jax.ShapeDtypeStruct((N, NCLASS), jnp.float32),
    )(parts, parts)


def kernel(x, src, tgt, Mtgt, W1, b1, W2, b2):
    npad = E_PAD - E
    pad_idx = jnp.arange(npad, dtype=jnp.int32) % N
    srca = jnp.concatenate([src.astype(jnp.int32), pad_idx])
    srcb = srca + N
    tgt1 = jnp.concatenate([tgt.astype(jnp.int32), pad_idx])
    m1 = jnp.concatenate([Mtgt, jnp.zeros((npad,), jnp.float32)])

    w1st = jnp.concatenate([W1[:, :HALF], W1[:, HALF:]], axis=0)
    b1st = jnp.stack([b1[:HALF], b1[HALF:]]).reshape(2, 1, HALF)
    s1 = _mm_split(x, w1st, b1st)
    p1 = _get_sc_l1()(s1, srca, srcb, tgt1, m1)
    ha, s2h = _l2(p1, W2[:, :HALF], b2[:HALF].reshape(1, HALF))
    p2 = _get_sc_l2()(s2h, srca, tgt1, m1, ha)
    return _final(p2)
